# baseline (speedup 1.0000x reference)
import functools

import jax
import jax.numpy as jnp
from jax import lax
from jax.experimental import pallas as pl
from jax.experimental.pallas import tpu as pltpu
from jax.experimental.pallas import tpu_sc as plsc

N = 50000
E = 800000
G = 1024
NP = 50048
NH = 25024
NH_P = 25088
TRASH_H = 25080
EP = 819200
SB = 512
NB = 53248
GP = 1152
ROW1 = 3128
ROWH = 1568
ROWG = 72
NSB1 = 50
NSB2 = 100
NBK = 13
NBK2 = 26


def _mesh():
    return plsc.VectorSubcoreMesh(core_axis_name="c", subcore_axis_name="s")


_CP = pltpu.CompilerParams(use_tc_tiling_on_sc=False)


def _zero_rows(zbuf, acc, base, nrows):
    done = 0
    while done < nrows:
        step = min(1000, nrows - done)
        pltpu.sync_copy(zbuf.at[pl.ds(0, step)], acc.at[pl.ds(base + done, step)])
        done += step


def _copy_rows(acc, base, nrows, zbuf, out_ref, obase):
    done = 0
    while done < nrows:
        step = min(1000, nrows - done)
        pltpu.sync_copy(acc.at[pl.ds(base + done, step)], zbuf.at[pl.ds(0, step)])
        pltpu.sync_copy(zbuf.at[pl.ds(0, step)], out_ref.at[pl.ds(obase + done, step)])
        done += step


def _edge_pipeline(nsb, brow0, sd_hbm, acc, sdbuf, dstbuf, sems_ix, sems_sc,
                   compute_dst, fire_scatters, drain_scatters,
                   fixbuf=None, rows=None, tbl_hbm=None, sems_g=None, off=None):
    gather = tbl_hbm is not None

    def idx_start(c, p):
        pltpu.async_copy(sd_hbm.at[pl.ds(brow0 + c * 4, 4)], sdbuf[p], sems_ix[p])

    def idx_drain(p):
        pltpu.make_async_copy(sd_hbm.at[pl.ds(brow0, 4)], sdbuf[p], sems_ix[p]).wait()

    def compute(p):
        for j in range(4):
            for kk in range(8):
                sl = pl.ds(kk * 16, 16)
                if gather:
                    fixbuf[p][j, sl] = sdbuf[p][j, 0, sl] + off
                compute_dst(dstbuf[p], sdbuf[p], j, sl)

    def gather_fire(p):
        for j in range(4):
            pltpu.async_copy(tbl_hbm.at[fixbuf[p].at[j]],
                             rows[p].at[pl.ds(j * 128, 128)], sems_g[p])

    def gather_drain(p):
        pltpu.make_async_copy(tbl_hbm.at[pl.ds(0, SB)], rows[p], sems_g[p]).wait()

    def stage_front(i, c, p):
        @pl.when(i > 0)
        def _():
            drain_scatters(p)

        idx_drain(p)
        compute(p)

        @pl.when(c + 2 < nsb)
        def _():
            idx_start(c + 2, p)

        if gather:
            gather_fire(p)

    def body(i, carry):
        a = 2 * i
        stage_front(i, a, 0)
        stage_front(i, a + 1, 1)
        if gather:
            gather_drain(0)
        fire_scatters(0)
        if gather:
            gather_drain(1)
        fire_scatters(1)
        return carry

    idx_start(0, 0)
    idx_start(1, 1)
    lax.fori_loop(0, nsb // 2, body, 0)
    drain_scatters(0)
    drain_scatters(1)


def _deg_counts(sd, batch_p, ones_blk, ones8_blk, z16, z8):
    @functools.partial(
        pl.kernel,
        out_type=(
            jax.ShapeDtypeStruct((2, NH_P, 8), jnp.float32),
            jax.ShapeDtypeStruct((2, GP, 16), jnp.float32),
        ),
        mesh=_mesh(),
        compiler_params=_CP,
        scratch_types=[
            pltpu.VMEM((1000, 16), jnp.float32),
            pltpu.VMEM((ROWH, 8), jnp.float32),
            pltpu.VMEM((128, 16), jnp.float32),
            pltpu.VMEM((128, 8), jnp.float32),
            pltpu.VMEM((128,), jnp.int32),
            pltpu.VMEM((4, 2, 128), jnp.int32),
            pltpu.VMEM((4, 2, 128), jnp.int32),
            pltpu.VMEM((4, 128), jnp.int32),
            pltpu.VMEM((4, 128), jnp.int32),
            pltpu.SemaphoreType.DMA,
            pltpu.SemaphoreType.DMA,
            pltpu.SemaphoreType.DMA,
            pltpu.SemaphoreType.DMA,
            pltpu.VMEM_SHARED((NH_P, 8), jnp.float32),
            pltpu.VMEM_SHARED((GP, 16), jnp.float32),
        ],
    )
    def k(sd_hbm, batch_hbm, ones_hbm, ones8_hbm, z16_hbm, z8_hbm,
          deg_out, cnt_out,
          zbuf, zbuf8, ones_v, ones8, dix, sdb0, sdb1, db0, db1,
          ix0, ix1, sc0, sc1, accd, accc):
        c = lax.axis_index("c")
        s = lax.axis_index("s")
        w = c * 16 + s
        base = c * NH
        pltpu.sync_copy(z16_hbm, zbuf)
        pltpu.sync_copy(z8_hbm, zbuf8)
        pltpu.sync_copy(zbuf8, accd.at[pl.ds(s * ROWH, ROWH)])
        pltpu.sync_copy(zbuf.at[pl.ds(0, ROWG)], accc.at[pl.ds(s * ROWG, ROWG)])
        pltpu.sync_copy(ones_hbm, ones_v)
        pltpu.sync_copy(ones8_hbm, ones8)
        plsc.subcore_barrier()

        dstbuf = (db0, db1)
        sems_sc = (sc0, sc1)

        def compute_dst(db, sdb, j, sl):
            v = sdb[j, 1, sl] - base
            ok = jnp.logical_and(v >= 0, v < NH)
            db[j, sl] = jnp.where(ok, v, TRASH_H)

        def fire_scatters(p):
            for j in range(4):
                pltpu.async_copy(ones8, accd.at[dstbuf[p].at[j]],
                                 sems_sc[p], add=True)

        def drain_scatters(p):
            for j in range(4):
                pltpu.make_async_copy(ones8, accd.at[pl.ds(0, 128)],
                                      sems_sc[p]).wait()

        _edge_pipeline(NSB2, s * 400, sd_hbm, accd, (sdb0, sdb1), dstbuf,
                       (ix0, ix1), sems_sc, compute_dst, fire_scatters,
                       drain_scatters)

        nb = w * (NBK * 128)

        def node_blk(b, carry):
            pltpu.sync_copy(batch_hbm.at[pl.ds(nb + b * 128, 128)], dix)
            pltpu.sync_copy(ones_v, accc.at[dix], add=True)
            return carry

        lax.fori_loop(0, NBK, node_blk, 0)
        plsc.subcore_barrier()
        pltpu.sync_copy(accd.at[pl.ds(s * ROWH, ROWH)], zbuf8)
        pltpu.sync_copy(zbuf8, deg_out.at[c, pl.ds(s * ROWH, ROWH)])
        pltpu.sync_copy(accc.at[pl.ds(s * ROWG, ROWG)], zbuf.at[pl.ds(0, ROWG)])
        pltpu.sync_copy(zbuf.at[pl.ds(0, ROWG)], cnt_out.at[c, pl.ds(s * ROWG, ROWG)])

    return k(sd, batch_p, ones_blk, ones8_blk, z16, z8)


def _conv_scratch():
    return [
        pltpu.VMEM((1000, 16), jnp.float32),
        pltpu.VMEM((4, 2, 128), jnp.int32),
        pltpu.VMEM((4, 2, 128), jnp.int32),
        pltpu.VMEM((4, 128), jnp.int32),
        pltpu.VMEM((4, 128), jnp.int32),
        pltpu.VMEM((4, 128), jnp.int32),
        pltpu.VMEM((4, 128), jnp.int32),
        pltpu.VMEM((SB, 16), jnp.float32),
        pltpu.VMEM((SB, 16), jnp.float32),
        pltpu.SemaphoreType.DMA,
        pltpu.SemaphoreType.DMA,
        pltpu.SemaphoreType.DMA,
        pltpu.SemaphoreType.DMA,
        pltpu.SemaphoreType.DMA,
        pltpu.SemaphoreType.DMA,
        pltpu.VMEM_SHARED((NP, 16), jnp.float32),
    ]


def _make_conv_helpers(acc, dstbuf, rows, sems_sc):
    def compute_dst(db, sdb, j, sl):
        db[j, sl] = sdb[j, 1, sl]

    def fire_scatters(p):
        for j in range(4):
            pltpu.async_copy(rows[p].at[pl.ds(j * 128, 128)],
                             acc.at[dstbuf[p].at[j]], sems_sc[p], add=True)

    def drain_scatters(p):
        pltpu.make_async_copy(rows[p], acc.at[pl.ds(0, SB)], sems_sc[p]).wait()

    return compute_dst, fire_scatters, drain_scatters


def _edge_scatter16(tbl, sd, z16):
    @functools.partial(
        pl.kernel,
        out_type=jax.ShapeDtypeStruct((2, NP, 16), jnp.float32),
        mesh=_mesh(),
        compiler_params=_CP,
        scratch_types=_conv_scratch(),
    )
    def k(tbl_hbm, sd_hbm, z16_hbm, p_out,
          zbuf, sdb0, sdb1, fb0, fb1, db0, db1, rows0, rows1,
          ix0, ix1, g0, g1, sc0, sc1, acc):
        c = lax.axis_index("c")
        s = lax.axis_index("s")
        w = c * 16 + s
        pltpu.sync_copy(z16_hbm, zbuf)
        _zero_rows(zbuf, acc, s * ROW1, ROW1)
        plsc.subcore_barrier()

        dstbuf = (db0, db1)
        rows = (rows0, rows1)
        sems_sc = (sc0, sc1)
        compute_dst, fire_scatters, drain_scatters = _make_conv_helpers(
            acc, dstbuf, rows, sems_sc)

        _edge_pipeline(NSB1, w * 200, sd_hbm, acc, (sdb0, sdb1), dstbuf,
                       (ix0, ix1), sems_sc, compute_dst, fire_scatters,
                       drain_scatters, fixbuf=(fb0, fb1), rows=rows,
                       tbl_hbm=tbl_hbm, sems_g=(g0, g1),
                       off=jnp.int32(0))

        plsc.subcore_barrier()
        _copy_rows(acc, s * ROW1, ROW1, zbuf, p_out.at[c], s * ROW1)

    return k(tbl, sd, z16)


def _edge_scatter64(tbl4, sd, z16):
    @functools.partial(
        pl.kernel,
        out_type=jax.ShapeDtypeStruct((4, NP, 16), jnp.float32),
        mesh=_mesh(),
        compiler_params=_CP,
        scratch_types=_conv_scratch(),
    )
    def k(tbl_hbm, sd_hbm, z16_hbm, s_out,
          zbuf, sdb0, sdb1, fb0, fb1, db0, db1, rows0, rows1,
          ix0, ix1, g0, g1, sc0, sc1, acc):
        c = lax.axis_index("c")
        s = lax.axis_index("s")

        dstbuf = (db0, db1)
        rows = (rows0, rows1)
        sems_sc = (sc0, sc1)
        compute_dst, fire_scatters, drain_scatters = _make_conv_helpers(
            acc, dstbuf, rows, sems_sc)

        for q in range(2):
            chunk = 2 * c + q
            pltpu.sync_copy(z16_hbm, zbuf)
            _zero_rows(zbuf, acc, s * ROW1, ROW1)
            plsc.subcore_barrier()

            _edge_pipeline(NSB2, s * 400, sd_hbm, acc, (sdb0, sdb1), dstbuf,
                           (ix0, ix1), sems_sc, compute_dst, fire_scatters,
                           drain_scatters, fixbuf=(fb0, fb1), rows=rows,
                           tbl_hbm=tbl_hbm, sems_g=(g0, g1),
                           off=chunk * NP)

            plsc.subcore_barrier()
            _copy_rows(acc, s * ROW1, ROW1, zbuf, s_out.at[chunk], s * ROW1)
            plsc.subcore_barrier()

    return k(tbl4, sd, z16)


def _pool_scatter(z2h, batch_p, z32):
    @functools.partial(
        pl.kernel,
        out_type=jax.ShapeDtypeStruct((2, GP, 32), jnp.float32),
        mesh=_mesh(),
        compiler_params=_CP,
        scratch_types=[
            pltpu.VMEM((ROWG, 32), jnp.float32),
            pltpu.VMEM((128, 32), jnp.float32),
            pltpu.VMEM((128,), jnp.int32),
            pltpu.VMEM_SHARED((GP, 32), jnp.float32),
        ],
    )
    def k(z2_hbm, batch_hbm, z32_hbm, p_out, zbuf, rows, dix, acc):
        c = lax.axis_index("c")
        s = lax.axis_index("s")
        pltpu.sync_copy(z32_hbm, zbuf)
        pltpu.sync_copy(zbuf, acc.at[pl.ds(s * ROWG, ROWG)])
        plsc.subcore_barrier()
        nb = s * (NBK2 * 128)

        def node_blk(b, carry):
            pltpu.sync_copy(z2_hbm.at[c, pl.ds(nb + b * 128, 128)], rows)
            pltpu.sync_copy(batch_hbm.at[pl.ds(nb + b * 128, 128)], dix)
            pltpu.sync_copy(rows, acc.at[dix], add=True)
            return carry

        lax.fori_loop(0, NBK2, node_blk, 0)
        plsc.subcore_barrier()
        pltpu.sync_copy(acc.at[pl.ds(s * ROWG, ROWG)], zbuf)
        pltpu.sync_copy(zbuf, p_out.at[c, pl.ds(s * ROWG, ROWG)])

    return k(z2h, batch_p, z32)


def kernel(x, edge_index, batch, W1, b1, W2, b2, Wl, bl):
    src = edge_index[0].astype(jnp.int32)
    dst = edge_index[1].astype(jnp.int32)
    batch = batch.astype(jnp.int32)

    src_p = jnp.concatenate([src, jnp.zeros((EP - E,), jnp.int32)])
    dst_p = jnp.concatenate([dst, jnp.full((EP - E,), N, jnp.int32)])
    sd = jnp.stack([src_p.reshape(-1, 128), dst_p.reshape(-1, 128)], axis=1)
    batch_p = jnp.concatenate([batch, jnp.full((NB - N,), G, jnp.int32)])
    ones_blk = jnp.ones((128, 16), jnp.float32)
    ones8_blk = jnp.ones((128, 8), jnp.float32)
    z16 = jnp.zeros((1000, 16), jnp.float32)
    z8 = jnp.zeros((ROWH, 8), jnp.float32)
    z32 = jnp.zeros((ROWG, 32), jnp.float32)

    deg_p, cnt_p = _deg_counts(sd, batch_p, ones_blk, ones8_blk, z16, z8)
    deg = jnp.concatenate([deg_p[0, :NH, 0], deg_p[1, : N - NH, 0]]) + 1.0
    counts = cnt_p[0, :G, 0] + cnt_p[1, :G, 0]
    dis = deg ** -0.5
    dis2 = dis * dis

    y0 = jnp.pad(x * dis[:, None], ((0, NP - N), (0, 16 - x.shape[1])))
    p0 = _edge_scatter16(y0, sd, z16)
    s0 = (p0[0] + p0[1])[:N, : x.shape[1]]
    z1 = dis[:, None] * s0 + dis2[:, None] * x

    h1c = jax.nn.relu(
        jnp.einsum("nk,kcj->cnj", z1, W1.reshape(W1.shape[0], 4, 16))
        + b1.reshape(4, 1, 16))
    y1c = jnp.pad(h1c * dis[None, :, None], ((0, 0), (0, NP - N), (0, 0)))
    s1c = _edge_scatter64(y1c.reshape(4 * NP, 16), sd, z16)
    z2c = dis[None, :, None] * s1c[:, :N, :] + dis2[None, :, None] * h1c

    z2cp = jnp.pad(z2c, ((0, 0), (0, NB - N), (0, 0)))
    z2h = jnp.stack([jnp.concatenate([z2cp[0], z2cp[1]], axis=1),
                     jnp.concatenate([z2cp[2], z2cp[3]], axis=1)])
    pp = _pool_scatter(z2h, batch_p, z32)
    sums = jnp.concatenate([pp[0, :G], pp[1, :G]], axis=1)
    g_pre = sums / jnp.clip(counts, 1.0)[:, None]
    W2c = W2.reshape(4, 16, 64)
    Wc = (W2c[0] @ Wl, W2c[1] @ Wl, W2c[2] @ Wl, W2c[3] @ Wl)
    out = g_pre[:, :16] @ Wc[0] + g_pre[:, 16:32] @ Wc[1] \
        + g_pre[:, 32:48] @ Wc[2] + g_pre[:, 48:] @ Wc[3] \
        + (counts > 0.0)[:, None] * (b2 @ Wl) + bl
    return out


# speedup vs baseline: 1.0208x; 1.0208x over previous
import functools

import jax
import jax.numpy as jnp
from jax import lax
from jax.experimental import pallas as pl
from jax.experimental.pallas import tpu as pltpu
from jax.experimental.pallas import tpu_sc as plsc

N = 50000
E = 800000
G = 1024
NP = 50048
NH = 25024
NH_P = 25088
TRASH_H = 25080
EP = 819200
SB = 512
NB = 53248
GP = 1152
ROW1 = 3128
ROWH = 1568
ROWG = 72
NSB1 = 50
NSB2 = 100
NBK = 13
NBK2 = 26


def _mesh():
    return plsc.VectorSubcoreMesh(core_axis_name="c", subcore_axis_name="s")


_CP = pltpu.CompilerParams(use_tc_tiling_on_sc=False)


def _zero_rows(zbuf, acc, base, nrows):
    cap = zbuf.shape[0]
    done = 0
    while done < nrows:
        step = min(cap, nrows - done)
        pltpu.sync_copy(zbuf.at[pl.ds(0, step)], acc.at[pl.ds(base + done, step)])
        done += step


def _copy_rows(acc, base, nrows, zbuf, out_ref, obase):
    cap = zbuf.shape[0]
    done = 0
    while done < nrows:
        step = min(cap, nrows - done)
        pltpu.sync_copy(acc.at[pl.ds(base + done, step)], zbuf.at[pl.ds(0, step)])
        pltpu.sync_copy(zbuf.at[pl.ds(0, step)], out_ref.at[pl.ds(obase + done, step)])
        done += step


def _edge_pipeline(nsb, brow0, sd_hbm, acc, sdbuf, dstbuf, sems_ix, sems_sc,
                   compute_dst, fire_scatters, drain_scatters,
                   fixbuf=None, rows=None, tbl_hbm=None, sems_g=None, off=None):
    gather = tbl_hbm is not None

    def idx_start(c, p):
        pltpu.async_copy(sd_hbm.at[pl.ds(brow0 + c * 4, 4)], sdbuf[p], sems_ix[p])

    def idx_drain(p):
        pltpu.make_async_copy(sd_hbm.at[pl.ds(brow0, 4)], sdbuf[p], sems_ix[p]).wait()

    def compute(p):
        for j in range(4):
            for kk in range(8):
                sl = pl.ds(kk * 16, 16)
                if gather:
                    fixbuf[p][j, sl] = sdbuf[p][j, 0, sl] + off
                compute_dst(dstbuf[p], sdbuf[p], j, sl)

    def gather_fire(p):
        for j in range(4):
            pltpu.async_copy(tbl_hbm.at[fixbuf[p].at[j]],
                             rows[p].at[pl.ds(j * 128, 128)], sems_g[p])

    def gather_drain(p):
        pltpu.make_async_copy(tbl_hbm.at[pl.ds(0, SB)], rows[p], sems_g[p]).wait()

    def stage_front(i, c, p):
        @pl.when(i > 0)
        def _():
            drain_scatters(p)

        idx_drain(p)
        compute(p)

        @pl.when(c + 2 < nsb)
        def _():
            idx_start(c + 2, p)

        if gather:
            gather_fire(p)

    def body(i, carry):
        a = 2 * i
        stage_front(i, a, 0)
        stage_front(i, a + 1, 1)
        if gather:
            gather_drain(0)
        fire_scatters(0)
        if gather:
            gather_drain(1)
        fire_scatters(1)
        return carry

    idx_start(0, 0)
    idx_start(1, 1)
    lax.fori_loop(0, nsb // 2, body, 0)
    drain_scatters(0)
    drain_scatters(1)


def _deg_counts(sd, batch_p, ones_blk, ones8_blk, z16, z8):
    @functools.partial(
        pl.kernel,
        out_type=(
            jax.ShapeDtypeStruct((2, NH_P, 8), jnp.float32),
            jax.ShapeDtypeStruct((2, GP, 16), jnp.float32),
        ),
        mesh=_mesh(),
        compiler_params=_CP,
        scratch_types=[
            pltpu.VMEM((504, 16), jnp.float32),
            pltpu.VMEM((784, 8), jnp.float32),
            pltpu.VMEM((128, 16), jnp.float32),
            pltpu.VMEM((128, 8), jnp.float32),
            pltpu.VMEM((128,), jnp.int32),
            pltpu.VMEM((4, 2, 128), jnp.int32),
            pltpu.VMEM((4, 2, 128), jnp.int32),
            pltpu.VMEM((4, 128), jnp.int32),
            pltpu.VMEM((4, 128), jnp.int32),
            pltpu.SemaphoreType.DMA,
            pltpu.SemaphoreType.DMA,
            pltpu.SemaphoreType.DMA,
            pltpu.SemaphoreType.DMA,
            pltpu.VMEM_SHARED((NH_P, 8), jnp.float32),
            pltpu.VMEM_SHARED((GP, 16), jnp.float32),
        ],
    )
    def k(sd_hbm, batch_hbm, ones_hbm, ones8_hbm, z16_hbm, z8_hbm,
          deg_out, cnt_out,
          zbuf, zbuf8, ones_v, ones8, dix, sdb0, sdb1, db0, db1,
          ix0, ix1, sc0, sc1, accd, accc):
        c = lax.axis_index("c")
        s = lax.axis_index("s")
        w = c * 16 + s
        base = c * NH
        pltpu.sync_copy(z16_hbm.at[pl.ds(0, 504)], zbuf)
        pltpu.sync_copy(z8_hbm.at[pl.ds(0, 784)], zbuf8)
        pltpu.sync_copy(zbuf8, accd.at[pl.ds(s * ROWH, 784)])
        pltpu.sync_copy(zbuf8, accd.at[pl.ds(s * ROWH + 784, 784)])
        pltpu.sync_copy(zbuf.at[pl.ds(0, ROWG)], accc.at[pl.ds(s * ROWG, ROWG)])
        pltpu.sync_copy(ones_hbm, ones_v)
        pltpu.sync_copy(ones8_hbm, ones8)
        plsc.subcore_barrier()

        dstbuf = (db0, db1)
        sems_sc = (sc0, sc1)

        def compute_dst(db, sdb, j, sl):
            v = sdb[j, 1, sl] - base
            ok = jnp.logical_and(v >= 0, v < NH)
            db[j, sl] = jnp.where(ok, v, TRASH_H)

        def fire_scatters(p):
            for j in range(4):
                pltpu.async_copy(ones8, accd.at[dstbuf[p].at[j]],
                                 sems_sc[p], add=True)

        def drain_scatters(p):
            for j in range(4):
                pltpu.make_async_copy(ones8, accd.at[pl.ds(0, 128)],
                                      sems_sc[p]).wait()

        _edge_pipeline(NSB2, s * 400, sd_hbm, accd, (sdb0, sdb1), dstbuf,
                       (ix0, ix1), sems_sc, compute_dst, fire_scatters,
                       drain_scatters)

        nb = w * (NBK * 128)

        def node_blk(b, carry):
            pltpu.sync_copy(batch_hbm.at[pl.ds(nb + b * 128, 128)], dix)
            pltpu.sync_copy(ones_v, accc.at[dix], add=True)
            return carry

        lax.fori_loop(0, NBK, node_blk, 0)
        plsc.subcore_barrier()
        for hh in range(2):
            pltpu.sync_copy(accd.at[pl.ds(s * ROWH + hh * 784, 784)], zbuf8)
            pltpu.sync_copy(zbuf8, deg_out.at[c, pl.ds(s * ROWH + hh * 784, 784)])
        pltpu.sync_copy(accc.at[pl.ds(s * ROWG, ROWG)], zbuf.at[pl.ds(0, ROWG)])
        pltpu.sync_copy(zbuf.at[pl.ds(0, ROWG)], cnt_out.at[c, pl.ds(s * ROWG, ROWG)])

    return k(sd, batch_p, ones_blk, ones8_blk, z16, z8)


def _conv_scratch():
    return [
        pltpu.VMEM((504, 16), jnp.float32),
        pltpu.VMEM((4, 2, 128), jnp.int32),
        pltpu.VMEM((4, 2, 128), jnp.int32),
        pltpu.VMEM((4, 128), jnp.int32),
        pltpu.VMEM((4, 128), jnp.int32),
        pltpu.VMEM((4, 128), jnp.int32),
        pltpu.VMEM((4, 128), jnp.int32),
        pltpu.VMEM((SB, 16), jnp.float32),
        pltpu.VMEM((SB, 16), jnp.float32),
        pltpu.SemaphoreType.DMA,
        pltpu.SemaphoreType.DMA,
        pltpu.SemaphoreType.DMA,
        pltpu.SemaphoreType.DMA,
        pltpu.SemaphoreType.DMA,
        pltpu.SemaphoreType.DMA,
        pltpu.VMEM_SHARED((NP, 16), jnp.float32),
    ]


def _make_conv_helpers(acc, dstbuf, rows, sems_sc):
    def compute_dst(db, sdb, j, sl):
        db[j, sl] = sdb[j, 1, sl]

    def fire_scatters(p):
        for j in range(4):
            pltpu.async_copy(rows[p].at[pl.ds(j * 128, 128)],
                             acc.at[dstbuf[p].at[j]], sems_sc[p], add=True)

    def drain_scatters(p):
        pltpu.make_async_copy(rows[p], acc.at[pl.ds(0, SB)], sems_sc[p]).wait()

    return compute_dst, fire_scatters, drain_scatters


def _edge_scatter16(tbl, sd, z16):
    @functools.partial(
        pl.kernel,
        out_type=jax.ShapeDtypeStruct((2, NP, 16), jnp.float32),
        mesh=_mesh(),
        compiler_params=_CP,
        scratch_types=_conv_scratch(),
    )
    def k(tbl_hbm, sd_hbm, z16_hbm, p_out,
          zbuf, sdb0, sdb1, fb0, fb1, db0, db1, rows0, rows1,
          ix0, ix1, g0, g1, sc0, sc1, acc):
        c = lax.axis_index("c")
        s = lax.axis_index("s")
        w = c * 16 + s
        pltpu.sync_copy(z16_hbm.at[pl.ds(0, 504)], zbuf)
        _zero_rows(zbuf, acc, s * ROW1, ROW1)
        plsc.subcore_barrier()

        dstbuf = (db0, db1)
        rows = (rows0, rows1)
        sems_sc = (sc0, sc1)
        compute_dst, fire_scatters, drain_scatters = _make_conv_helpers(
            acc, dstbuf, rows, sems_sc)

        _edge_pipeline(NSB1, w * 200, sd_hbm, acc, (sdb0, sdb1), dstbuf,
                       (ix0, ix1), sems_sc, compute_dst, fire_scatters,
                       drain_scatters, fixbuf=(fb0, fb1), rows=rows,
                       tbl_hbm=tbl_hbm, sems_g=(g0, g1),
                       off=jnp.int32(0))

        plsc.subcore_barrier()
        _copy_rows(acc, s * ROW1, ROW1, zbuf, p_out.at[c], s * ROW1)

    return k(tbl, sd, z16)


def _edge_scatter64(tbl4, sd, z16, disb, dh, batch_p):
    @functools.partial(
        pl.kernel,
        out_type=jax.ShapeDtypeStruct((4, GP, 16), jnp.float32),
        mesh=_mesh(),
        compiler_params=_CP,
        scratch_types=[
            pltpu.VMEM((1024, 16), jnp.float32),
            pltpu.VMEM((4, 2, 128), jnp.int32),
            pltpu.VMEM((4, 2, 128), jnp.int32),
            pltpu.VMEM((4, 128), jnp.int32),
            pltpu.VMEM((4, 128), jnp.int32),
            pltpu.VMEM((4, 128), jnp.int32),
            pltpu.VMEM((4, 128), jnp.int32),
            pltpu.VMEM((SB, 16), jnp.float32),
            pltpu.VMEM((SB, 16), jnp.float32),
            pltpu.VMEM((1024, 16), jnp.float32),
            pltpu.VMEM((1024, 16), jnp.float32),
            pltpu.VMEM((128,), jnp.int32),
            pltpu.VMEM((56,), jnp.int32),
            pltpu.SemaphoreType.DMA,
            pltpu.SemaphoreType.DMA,
            pltpu.SemaphoreType.DMA,
            pltpu.SemaphoreType.DMA,
            pltpu.SemaphoreType.DMA,
            pltpu.SemaphoreType.DMA,
            pltpu.VMEM_SHARED((NP, 16), jnp.float32),
            pltpu.VMEM_SHARED((GP, 16), jnp.float32),
            pltpu.VMEM_SHARED((GP, 16), jnp.float32),
        ],
    )
    def k(tbl_hbm, sd_hbm, z16_hbm, disb_hbm, dh_hbm, batch_hbm, pool_out,
          zbuf, sdb0, sdb1, fb0, fb1, db0, db1, rows0, rows1, disbuf, hbuf,
          dix, dix56, ix0, ix1, g0, g1, sc0, sc1, acc, accp0, accp1):
        c = lax.axis_index("c")
        s = lax.axis_index("s")

        dstbuf = (db0, db1)
        rows = (rows0, rows1)
        sems_sc = (sc0, sc1)
        accps = (accp0, accp1)
        compute_dst, fire_scatters, drain_scatters = _make_conv_helpers(
            acc, dstbuf, rows, sems_sc)

        pltpu.sync_copy(z16_hbm, zbuf)
        for q in range(2):
            pltpu.sync_copy(zbuf.at[pl.ds(0, ROWG)],
                            accps[q].at[pl.ds(s * ROWG, ROWG)])

        for q in range(2):
            chunk = 2 * c + q
            pltpu.sync_copy(z16_hbm, zbuf)
            _zero_rows(zbuf, acc, s * ROW1, ROW1)
            plsc.subcore_barrier()

            _edge_pipeline(NSB2, s * 400, sd_hbm, acc, (sdb0, sdb1), dstbuf,
                           (ix0, ix1), sems_sc, compute_dst, fire_scatters,
                           drain_scatters, fixbuf=(fb0, fb1), rows=rows,
                           tbl_hbm=tbl_hbm, sems_g=(g0, g1),
                           off=chunk * NP)

            plsc.subcore_barrier()
            node00 = s * ROW1
            done = 0
            for step in (1024, 1024, 1024, 56):
                base = node00 + done
                pltpu.sync_copy(acc.at[pl.ds(base, step)],
                                zbuf.at[pl.ds(0, step)])
                pltpu.sync_copy(disb_hbm.at[pl.ds(base, step)],
                                disbuf.at[pl.ds(0, step)])
                pltpu.sync_copy(dh_hbm.at[chunk, pl.ds(base, step)],
                                hbuf.at[pl.ds(0, step)])

                def zrow(r, carry):
                    zbuf[r, :] = zbuf[r, :] * disbuf[r, :] + hbuf[r, :]
                    return carry

                lax.fori_loop(0, step, zrow, 0)
                for b in range(step // 128):
                    pltpu.sync_copy(batch_hbm.at[pl.ds(base + b * 128, 128)],
                                    dix)
                    pltpu.sync_copy(zbuf.at[pl.ds(b * 128, 128)],
                                    accps[q].at[dix], add=True)
                if step % 128:
                    tail = step % 128
                    toff = (step // 128) * 128
                    pltpu.sync_copy(batch_hbm.at[pl.ds(base + toff, tail)],
                                    dix56)
                    pltpu.sync_copy(zbuf.at[pl.ds(toff, tail)],
                                    accps[q].at[dix56], add=True)
                done += step
            plsc.subcore_barrier()

        for q in range(2):
            pltpu.sync_copy(accps[q].at[pl.ds(s * ROWG, ROWG)],
                            zbuf.at[pl.ds(0, ROWG)])
            pltpu.sync_copy(zbuf.at[pl.ds(0, ROWG)],
                            pool_out.at[2 * c + q, pl.ds(s * ROWG, ROWG)])

    return k(tbl4, sd, z16, disb, dh, batch_p)


def kernel(x, edge_index, batch, W1, b1, W2, b2, Wl, bl):
    src = edge_index[0].astype(jnp.int32)
    dst = edge_index[1].astype(jnp.int32)
    batch = batch.astype(jnp.int32)

    src_p = jnp.concatenate([src, jnp.zeros((EP - E,), jnp.int32)])
    dst_p = jnp.concatenate([dst, jnp.full((EP - E,), N, jnp.int32)])
    sd = jnp.stack([src_p.reshape(-1, 128), dst_p.reshape(-1, 128)], axis=1)
    batch_p = jnp.concatenate([batch, jnp.full((NB - N,), G, jnp.int32)])
    ones_blk = jnp.ones((128, 16), jnp.float32)
    ones8_blk = jnp.ones((128, 8), jnp.float32)
    z16 = jnp.zeros((1000, 16), jnp.float32)
    z16p = jnp.zeros((1024, 16), jnp.float32)
    z8 = jnp.zeros((784, 8), jnp.float32)

    deg_p, cnt_p = _deg_counts(sd, batch_p, ones_blk, ones8_blk, z16, z8)
    deg = jnp.concatenate([deg_p[0, :NH, 0], deg_p[1, : N - NH, 0]]) + 1.0
    counts = cnt_p[0, :G, 0] + cnt_p[1, :G, 0]
    dis = deg ** -0.5
    dis2 = dis * dis

    y0 = jnp.pad(x * dis[:, None], ((0, NP - N), (0, 16 - x.shape[1])))
    p0 = _edge_scatter16(y0, sd, z16)
    s0 = (p0[0] + p0[1])[:N, : x.shape[1]]
    z1 = dis[:, None] * s0 + dis2[:, None] * x

    h1c = jax.nn.relu(
        jnp.einsum("nk,kcj->cnj", z1, W1.reshape(W1.shape[0], 4, 16))
        + b1.reshape(4, 1, 16))
    y1c = jnp.pad(h1c * dis[None, :, None], ((0, 0), (0, NP - N), (0, 0)))
    disb = jnp.pad(jnp.broadcast_to(dis[:, None], (N, 16)), ((0, NP - N), (0, 0)))
    dh = jnp.pad(dis2[None, :, None] * h1c, ((0, 0), (0, NP - N), (0, 0)))
    pools = _edge_scatter64(y1c.reshape(4 * NP, 16), sd, z16p, disb, dh,
                            batch_p)

    sums = jnp.concatenate([pools[k, :G] for k in range(4)], axis=1)
    g_pre = sums / jnp.clip(counts, 1.0)[:, None]
    out = g_pre @ (W2 @ Wl) + (counts > 0.0)[:, None] * (b2 @ Wl) + bl
    return out


# speedup vs baseline: 1.1782x; 1.1542x over previous
import functools

import jax
import jax.numpy as jnp
from jax import lax
from jax.experimental import pallas as pl
from jax.experimental.pallas import tpu as pltpu
from jax.experimental.pallas import tpu_sc as plsc

N = 50000
E = 800000
G = 1024
NP = 50048
NH = 25024
NH_P = 25088
TRASH_H = 25080
EP = 819200
SB = 512
NB = 53248
GP = 1152
ROW1 = 3128
ROWH = 1568
ROWG = 72
NSB1 = 50
NSB2 = 100
NBK = 13
NBK2 = 26


def _mesh():
    return plsc.VectorSubcoreMesh(core_axis_name="c", subcore_axis_name="s")


_CP = pltpu.CompilerParams(use_tc_tiling_on_sc=False)


def _zero_rows(zbuf, acc, base, nrows):
    cap = zbuf.shape[0]
    done = 0
    while done < nrows:
        step = min(cap, nrows - done)
        pltpu.sync_copy(zbuf.at[pl.ds(0, step)], acc.at[pl.ds(base + done, step)])
        done += step


def _copy_rows(acc, base, nrows, zbuf, out_ref, obase):
    cap = zbuf.shape[0]
    done = 0
    while done < nrows:
        step = min(cap, nrows - done)
        pltpu.sync_copy(acc.at[pl.ds(base + done, step)], zbuf.at[pl.ds(0, step)])
        pltpu.sync_copy(zbuf.at[pl.ds(0, step)], out_ref.at[pl.ds(obase + done, step)])
        done += step


def _edge_pipeline(nsb, brow0, sd_hbm, acc, sdbuf, dstbuf, sems_ix, sems_sc,
                   compute_dst, fire_scatters, drain_scatters,
                   fixbuf=None, rows=None, tbl_hbm=None, sems_g=None, off=None):
    gather = tbl_hbm is not None

    def idx_start(c, p):
        pltpu.async_copy(sd_hbm.at[pl.ds(brow0 + c * 4, 4)], sdbuf[p], sems_ix[p])

    def idx_drain(p):
        pltpu.make_async_copy(sd_hbm.at[pl.ds(brow0, 4)], sdbuf[p], sems_ix[p]).wait()

    def compute(p):
        for j in range(4):
            for kk in range(8):
                sl = pl.ds(kk * 16, 16)
                if gather:
                    fixbuf[p][j, sl] = sdbuf[p][j, 0, sl] + off
                compute_dst(dstbuf[p], sdbuf[p], j, sl)

    def gather_fire(p):
        for j in range(4):
            pltpu.async_copy(tbl_hbm.at[fixbuf[p].at[j]],
                             rows[p].at[pl.ds(j * 128, 128)], sems_g[p])

    def gather_drain(p):
        pltpu.make_async_copy(tbl_hbm.at[pl.ds(0, SB)], rows[p], sems_g[p]).wait()

    def stage_front(i, c, p):
        @pl.when(i > 0)
        def _():
            drain_scatters(p)

        idx_drain(p)
        compute(p)

        @pl.when(c + 2 < nsb)
        def _():
            idx_start(c + 2, p)

        if gather:
            gather_fire(p)

    def body(i, carry):
        a = 2 * i
        stage_front(i, a, 0)
        stage_front(i, a + 1, 1)
        if gather:
            gather_drain(0)
        fire_scatters(0)
        if gather:
            gather_drain(1)
        fire_scatters(1)
        return carry

    idx_start(0, 0)
    idx_start(1, 1)
    lax.fori_loop(0, nsb // 2, body, 0)
    drain_scatters(0)
    drain_scatters(1)


def _deg_counts(sd, batch_p, ones_blk, ones8_blk, z16, z8):
    @functools.partial(
        pl.kernel,
        out_type=(
            jax.ShapeDtypeStruct((2, NH_P, 8), jnp.float32),
            jax.ShapeDtypeStruct((2, GP, 16), jnp.float32),
        ),
        mesh=_mesh(),
        compiler_params=_CP,
        scratch_types=[
            pltpu.VMEM((504, 16), jnp.float32),
            pltpu.VMEM((784, 8), jnp.float32),
            pltpu.VMEM((128, 16), jnp.float32),
            pltpu.VMEM((128, 8), jnp.float32),
            pltpu.VMEM((128,), jnp.int32),
            pltpu.VMEM((4, 2, 128), jnp.int32),
            pltpu.VMEM((4, 2, 128), jnp.int32),
            pltpu.VMEM((4, 128), jnp.int32),
            pltpu.VMEM((4, 128), jnp.int32),
            pltpu.SemaphoreType.DMA,
            pltpu.SemaphoreType.DMA,
            pltpu.SemaphoreType.DMA,
            pltpu.SemaphoreType.DMA,
            pltpu.VMEM_SHARED((NH_P, 8), jnp.float32),
            pltpu.VMEM_SHARED((GP, 16), jnp.float32),
        ],
    )
    def k(sd_hbm, batch_hbm, ones_hbm, ones8_hbm, z16_hbm, z8_hbm,
          deg_out, cnt_out,
          zbuf, zbuf8, ones_v, ones8, dix, sdb0, sdb1, db0, db1,
          ix0, ix1, sc0, sc1, accd, accc):
        c = lax.axis_index("c")
        s = lax.axis_index("s")
        w = c * 16 + s
        base = c * NH
        pltpu.sync_copy(z16_hbm.at[pl.ds(0, 504)], zbuf)
        pltpu.sync_copy(z8_hbm.at[pl.ds(0, 784)], zbuf8)
        pltpu.sync_copy(zbuf8, accd.at[pl.ds(s * ROWH, 784)])
        pltpu.sync_copy(zbuf8, accd.at[pl.ds(s * ROWH + 784, 784)])
        pltpu.sync_copy(zbuf.at[pl.ds(0, ROWG)], accc.at[pl.ds(s * ROWG, ROWG)])
        pltpu.sync_copy(ones_hbm, ones_v)
        pltpu.sync_copy(ones8_hbm, ones8)
        plsc.subcore_barrier()

        dstbuf = (db0, db1)
        sems_sc = (sc0, sc1)

        def compute_dst(db, sdb, j, sl):
            v = sdb[j, 1, sl] - base
            ok = jnp.logical_and(v >= 0, v < NH)
            db[j, sl] = jnp.where(ok, v, TRASH_H)

        def fire_scatters(p):
            for j in range(4):
                pltpu.async_copy(ones8, accd.at[dstbuf[p].at[j]],
                                 sems_sc[p], add=True)

        def drain_scatters(p):
            for j in range(4):
                pltpu.make_async_copy(ones8, accd.at[pl.ds(0, 128)],
                                      sems_sc[p]).wait()

        _edge_pipeline(NSB2, s * 400, sd_hbm, accd, (sdb0, sdb1), dstbuf,
                       (ix0, ix1), sems_sc, compute_dst, fire_scatters,
                       drain_scatters)

        nb = w * (NBK * 128)

        def node_blk(b, carry):
            pltpu.sync_copy(batch_hbm.at[pl.ds(nb + b * 128, 128)], dix)
            pltpu.sync_copy(ones_v, accc.at[dix], add=True)
            return carry

        lax.fori_loop(0, NBK, node_blk, 0)
        plsc.subcore_barrier()
        for hh in range(2):
            pltpu.sync_copy(accd.at[pl.ds(s * ROWH + hh * 784, 784)], zbuf8)
            pltpu.sync_copy(zbuf8, deg_out.at[c, pl.ds(s * ROWH + hh * 784, 784)])
        pltpu.sync_copy(accc.at[pl.ds(s * ROWG, ROWG)], zbuf.at[pl.ds(0, ROWG)])
        pltpu.sync_copy(zbuf.at[pl.ds(0, ROWG)], cnt_out.at[c, pl.ds(s * ROWG, ROWG)])

    return k(sd, batch_p, ones_blk, ones8_blk, z16, z8)


def _conv_scratch():
    return [
        pltpu.VMEM((504, 16), jnp.float32),
        pltpu.VMEM((4, 2, 128), jnp.int32),
        pltpu.VMEM((4, 2, 128), jnp.int32),
        pltpu.VMEM((4, 128), jnp.int32),
        pltpu.VMEM((4, 128), jnp.int32),
        pltpu.VMEM((4, 128), jnp.int32),
        pltpu.VMEM((4, 128), jnp.int32),
        pltpu.VMEM((SB, 16), jnp.float32),
        pltpu.VMEM((SB, 16), jnp.float32),
        pltpu.SemaphoreType.DMA,
        pltpu.SemaphoreType.DMA,
        pltpu.SemaphoreType.DMA,
        pltpu.SemaphoreType.DMA,
        pltpu.SemaphoreType.DMA,
        pltpu.SemaphoreType.DMA,
        pltpu.VMEM_SHARED((NP, 16), jnp.float32),
    ]


def _make_conv_helpers(acc, dstbuf, rows, sems_sc):
    def compute_dst(db, sdb, j, sl):
        db[j, sl] = sdb[j, 1, sl]

    def fire_scatters(p):
        for j in range(4):
            pltpu.async_copy(rows[p].at[pl.ds(j * 128, 128)],
                             acc.at[dstbuf[p].at[j]], sems_sc[p], add=True)

    def drain_scatters(p):
        pltpu.make_async_copy(rows[p], acc.at[pl.ds(0, SB)], sems_sc[p]).wait()

    return compute_dst, fire_scatters, drain_scatters


def _edge_scatter16(tbl, sd, z16):
    @functools.partial(
        pl.kernel,
        out_type=jax.ShapeDtypeStruct((2, NP, 16), jnp.float32),
        mesh=_mesh(),
        compiler_params=_CP,
        scratch_types=_conv_scratch(),
    )
    def k(tbl_hbm, sd_hbm, z16_hbm, p_out,
          zbuf, sdb0, sdb1, fb0, fb1, db0, db1, rows0, rows1,
          ix0, ix1, g0, g1, sc0, sc1, acc):
        c = lax.axis_index("c")
        s = lax.axis_index("s")
        w = c * 16 + s
        pltpu.sync_copy(z16_hbm.at[pl.ds(0, 504)], zbuf)
        _zero_rows(zbuf, acc, s * ROW1, ROW1)
        plsc.subcore_barrier()

        dstbuf = (db0, db1)
        rows = (rows0, rows1)
        sems_sc = (sc0, sc1)
        compute_dst, fire_scatters, drain_scatters = _make_conv_helpers(
            acc, dstbuf, rows, sems_sc)

        _edge_pipeline(NSB1, w * 200, sd_hbm, acc, (sdb0, sdb1), dstbuf,
                       (ix0, ix1), sems_sc, compute_dst, fire_scatters,
                       drain_scatters, fixbuf=(fb0, fb1), rows=rows,
                       tbl_hbm=tbl_hbm, sems_g=(g0, g1),
                       off=jnp.int32(0))

        plsc.subcore_barrier()
        _copy_rows(acc, s * ROW1, ROW1, zbuf, p_out.at[c], s * ROW1)

    return k(tbl, sd, z16)


def _edge_scatter64(tbl4, sd, z16, disb, dh, batch_p):
    @functools.partial(
        pl.kernel,
        out_type=jax.ShapeDtypeStruct((4, GP, 16), jnp.float32),
        mesh=_mesh(),
        compiler_params=_CP,
        scratch_types=[
            pltpu.VMEM((1024, 16), jnp.float32),
            pltpu.VMEM((4, 2, 128), jnp.int32),
            pltpu.VMEM((4, 2, 128), jnp.int32),
            pltpu.VMEM((4, 128), jnp.int32),
            pltpu.VMEM((4, 128), jnp.int32),
            pltpu.VMEM((4, 128), jnp.int32),
            pltpu.VMEM((4, 128), jnp.int32),
            pltpu.VMEM((SB, 16), jnp.float32),
            pltpu.VMEM((SB, 16), jnp.float32),
            pltpu.VMEM((1024, 16), jnp.float32),
            pltpu.VMEM((1024, 16), jnp.float32),
            pltpu.VMEM((128,), jnp.int32),
            pltpu.VMEM((56,), jnp.int32),
            pltpu.SemaphoreType.DMA,
            pltpu.SemaphoreType.DMA,
            pltpu.SemaphoreType.DMA,
            pltpu.SemaphoreType.DMA,
            pltpu.SemaphoreType.DMA,
            pltpu.SemaphoreType.DMA,
            pltpu.VMEM_SHARED((NP, 16), jnp.float32),
            pltpu.VMEM_SHARED((GP, 16), jnp.float32),
            pltpu.VMEM_SHARED((GP, 16), jnp.float32),
        ],
    )
    def k(tbl_hbm, sd_hbm, z16_hbm, disb_hbm, dh_hbm, batch_hbm, pool_out,
          zbuf, sdb0, sdb1, fb0, fb1, db0, db1, rows0, rows1, disbuf, hbuf,
          dix, dix56, ix0, ix1, g0, g1, sc0, sc1, acc, accp0, accp1):
        c = lax.axis_index("c")
        s = lax.axis_index("s")

        dstbuf = (db0, db1)
        rows = (rows0, rows1)
        sems_sc = (sc0, sc1)
        accps = (accp0, accp1)
        compute_dst, fire_scatters, drain_scatters = _make_conv_helpers(
            acc, dstbuf, rows, sems_sc)

        pltpu.sync_copy(z16_hbm, zbuf)
        for q in range(2):
            pltpu.sync_copy(zbuf.at[pl.ds(0, ROWG)],
                            accps[q].at[pl.ds(s * ROWG, ROWG)])

        for q in range(2):
            chunk = 2 * c + q
            pltpu.sync_copy(z16_hbm, zbuf)
            _zero_rows(zbuf, acc, s * ROW1, ROW1)
            plsc.subcore_barrier()

            _edge_pipeline(NSB2, s * 400, sd_hbm, acc, (sdb0, sdb1), dstbuf,
                           (ix0, ix1), sems_sc, compute_dst, fire_scatters,
                           drain_scatters, fixbuf=(fb0, fb1), rows=rows,
                           tbl_hbm=tbl_hbm, sems_g=(g0, g1),
                           off=chunk * NP)

            plsc.subcore_barrier()
            node00 = s * ROW1
            done = 0
            for step in (1024, 1024, 1024, 56):
                base = node00 + done
                pltpu.sync_copy(acc.at[pl.ds(base, step)],
                                zbuf.at[pl.ds(0, step)])
                pltpu.sync_copy(disb_hbm.at[pl.ds(base, step)],
                                disbuf.at[pl.ds(0, step)])
                pltpu.sync_copy(dh_hbm.at[chunk, pl.ds(base, step)],
                                hbuf.at[pl.ds(0, step)])

                def zrow(r, carry):
                    zbuf[r, :] = zbuf[r, :] * disbuf[r, :] + hbuf[r, :]
                    return carry

                lax.fori_loop(0, step, zrow, 0)
                for b in range(step // 128):
                    pltpu.sync_copy(batch_hbm.at[pl.ds(base + b * 128, 128)],
                                    dix)
                    pltpu.sync_copy(zbuf.at[pl.ds(b * 128, 128)],
                                    accps[q].at[dix], add=True)
                if step % 128:
                    tail = step % 128
                    toff = (step // 128) * 128
                    pltpu.sync_copy(batch_hbm.at[pl.ds(base + toff, tail)],
                                    dix56)
                    pltpu.sync_copy(zbuf.at[pl.ds(toff, tail)],
                                    accps[q].at[dix56], add=True)
                done += step
            plsc.subcore_barrier()

        for q in range(2):
            pltpu.sync_copy(accps[q].at[pl.ds(s * ROWG, ROWG)],
                            zbuf.at[pl.ds(0, ROWG)])
            pltpu.sync_copy(zbuf.at[pl.ds(0, ROWG)],
                            pool_out.at[2 * c + q, pl.ds(s * ROWG, ROWG)])

    return k(tbl4, sd, z16, disb, dh, batch_p)


def kernel(x, edge_index, batch, W1, b1, W2, b2, Wl, bl):
    src = edge_index[0].astype(jnp.int32)
    dst = edge_index[1].astype(jnp.int32)
    batch = batch.astype(jnp.int32)

    src_p = jnp.concatenate([src, jnp.zeros((EP - E,), jnp.int32)])
    dst_p = jnp.concatenate([dst, jnp.full((EP - E,), N, jnp.int32)])
    sd = jnp.stack([src_p.reshape(-1, 128), dst_p.reshape(-1, 128)], axis=1)
    batch_p = jnp.concatenate([batch, jnp.full((NB - N,), G, jnp.int32)])
    ones_blk = jnp.ones((128, 16), jnp.float32)
    ones8_blk = jnp.ones((128, 8), jnp.float32)
    z16 = jnp.zeros((1000, 16), jnp.float32)
    z16p = jnp.zeros((1024, 16), jnp.float32)
    z8 = jnp.zeros((784, 8), jnp.float32)

    deg_p, cnt_p = _deg_counts(sd, batch_p, ones_blk, ones8_blk, z16, z8)
    deg = jnp.concatenate([deg_p[0, :NH, 0], deg_p[1, : N - NH, 0]]) + 1.0
    counts = cnt_p[0, :G, 0] + cnt_p[1, :G, 0]
    dis = deg ** -0.5
    dis2 = dis * dis

    y0 = jnp.pad(x * dis[:, None], ((0, NP - N), (0, 16 - x.shape[1])))
    p0 = _edge_scatter16(y0, sd, z16)
    s0 = (p0[0] + p0[1])[:N, : x.shape[1]]
    z1 = dis[:, None] * s0 + dis2[:, None] * x

    h1 = jax.nn.relu(z1 @ W1 + b1)
    y1 = jnp.pad(h1 * dis[:, None], ((0, NP - N), (0, 0)))
    tbl4 = y1.reshape(NP, 4, 16).transpose(1, 0, 2).reshape(4 * NP, 16)
    disb = jnp.pad(jnp.broadcast_to(dis[:, None], (N, 16)), ((0, NP - N), (0, 0)))
    dh = jnp.pad(dis2[:, None] * h1, ((0, NP - N), (0, 0)))
    dh = dh.reshape(NP, 4, 16).transpose(1, 0, 2)
    pools = _edge_scatter64(tbl4, sd, z16p, disb, dh, batch_p)

    sums = jnp.concatenate([pools[k, :G] for k in range(4)], axis=1)
    g_pre = sums / jnp.clip(counts, 1.0)[:, None]
    out = g_pre @ (W2 @ Wl) + (counts > 0.0)[:, None] * (b2 @ Wl) + bl
    return out


# speedup vs baseline: 1.5799x; 1.3409x over previous
import functools

import jax
import jax.numpy as jnp
from jax import lax
from jax.experimental import pallas as pl
from jax.experimental.pallas import tpu as pltpu
from jax.experimental.pallas import tpu_sc as plsc

N = 50000
E = 800000
G = 1024
NP = 50048
NH = 25024
NH_P = 25088
TRASH_H = 25080
EP = 819200
SB = 512
NB = 53248
GP = 1152
ROW1 = 3128
ROWH = 1568
ROWG = 72
NSB1 = 50
NSB2 = 100
NBK = 13
NBK2 = 26


def _mesh():
    return plsc.VectorSubcoreMesh(core_axis_name="c", subcore_axis_name="s")


_CP = pltpu.CompilerParams(use_tc_tiling_on_sc=False)


def _zero_rows(zbuf, acc, base, nrows):
    cap = zbuf.shape[0]
    done = 0
    while done < nrows:
        step = min(cap, nrows - done)
        pltpu.sync_copy(zbuf.at[pl.ds(0, step)], acc.at[pl.ds(base + done, step)])
        done += step


def _copy_rows(acc, base, nrows, zbuf, out_ref, obase):
    cap = zbuf.shape[0]
    done = 0
    while done < nrows:
        step = min(cap, nrows - done)
        pltpu.sync_copy(acc.at[pl.ds(base + done, step)], zbuf.at[pl.ds(0, step)])
        pltpu.sync_copy(zbuf.at[pl.ds(0, step)], out_ref.at[pl.ds(obase + done, step)])
        done += step


def _edge_pipeline(nsb, brow0, sd_hbm, acc, sdbuf, dstbuf, sems_ix, sems_sc,
                   compute_dst, fire_scatters, drain_scatters,
                   fixbuf=None, rows=None, tbl_hbm=None, sems_g=None, off=None):
    gather = tbl_hbm is not None

    def idx_start(c, p):
        pltpu.async_copy(sd_hbm.at[pl.ds(brow0 + c * 4, 4)], sdbuf[p], sems_ix[p])

    def idx_drain(p):
        pltpu.make_async_copy(sd_hbm.at[pl.ds(brow0, 4)], sdbuf[p], sems_ix[p]).wait()

    def compute(p):
        for j in range(4):
            for kk in range(8):
                sl = pl.ds(kk * 16, 16)
                if gather:
                    fixbuf[p][j, sl] = sdbuf[p][j, 0, sl] + off
                compute_dst(dstbuf[p], sdbuf[p], j, sl)

    def gather_fire(p):
        for j in range(4):
            pltpu.async_copy(tbl_hbm.at[fixbuf[p].at[j]],
                             rows[p].at[pl.ds(j * 128, 128)], sems_g[p])

    def gather_drain(p):
        pltpu.make_async_copy(tbl_hbm.at[pl.ds(0, SB)], rows[p], sems_g[p]).wait()

    def stage_front(i, c, p):
        @pl.when(i > 0)
        def _():
            drain_scatters(p)

        idx_drain(p)
        compute(p)

        @pl.when(c + 2 < nsb)
        def _():
            idx_start(c + 2, p)

        if gather:
            gather_fire(p)

    def body(i, carry):
        a = 2 * i
        stage_front(i, a, 0)
        stage_front(i, a + 1, 1)
        if gather:
            gather_drain(0)
        fire_scatters(0)
        if gather:
            gather_drain(1)
        fire_scatters(1)
        return carry

    idx_start(0, 0)
    idx_start(1, 1)
    lax.fori_loop(0, nsb // 2, body, 0)
    drain_scatters(0)
    drain_scatters(1)


def _deg_counts(sd, batch_p, ones_blk, ones8_blk, z16, z8):
    @functools.partial(
        pl.kernel,
        out_type=(
            jax.ShapeDtypeStruct((2, NP, 8), jnp.float32),
            jax.ShapeDtypeStruct((2, GP, 16), jnp.float32),
        ),
        mesh=_mesh(),
        compiler_params=_CP,
        scratch_types=[
            pltpu.VMEM((504, 16), jnp.float32),
            pltpu.VMEM((784, 8), jnp.float32),
            pltpu.VMEM((128, 16), jnp.float32),
            pltpu.VMEM((128, 8), jnp.float32),
            pltpu.VMEM((128,), jnp.int32),
            pltpu.VMEM((4, 2, 128), jnp.int32),
            pltpu.VMEM((4, 2, 128), jnp.int32),
            pltpu.VMEM((4, 128), jnp.int32),
            pltpu.VMEM((4, 128), jnp.int32),
            pltpu.SemaphoreType.DMA,
            pltpu.SemaphoreType.DMA,
            pltpu.SemaphoreType.DMA,
            pltpu.SemaphoreType.DMA,
            pltpu.VMEM_SHARED((NP, 8), jnp.float32),
            pltpu.VMEM_SHARED((GP, 16), jnp.float32),
        ],
    )
    def k(sd_hbm, batch_hbm, ones_hbm, ones8_hbm, z16_hbm, z8_hbm,
          deg_out, cnt_out,
          zbuf, zbuf8, ones_v, ones8, dix, sdb0, sdb1, db0, db1,
          ix0, ix1, sc0, sc1, accd, accc):
        c = lax.axis_index("c")
        s = lax.axis_index("s")
        w = c * 16 + s
        pltpu.sync_copy(z16_hbm.at[pl.ds(0, 504)], zbuf)
        pltpu.sync_copy(z8_hbm.at[pl.ds(0, 784)], zbuf8)
        for step, off8 in ((784, 0), (784, 784), (784, 1568), (776, 2352)):
            pltpu.sync_copy(zbuf8.at[pl.ds(0, step)],
                            accd.at[pl.ds(s * ROW1 + off8, step)])
        pltpu.sync_copy(zbuf.at[pl.ds(0, ROWG)], accc.at[pl.ds(s * ROWG, ROWG)])
        pltpu.sync_copy(ones_hbm, ones_v)
        pltpu.sync_copy(ones8_hbm, ones8)
        plsc.subcore_barrier()

        dstbuf = (db0, db1)
        sems_sc = (sc0, sc1)

        def compute_dst(db, sdb, j, sl):
            db[j, sl] = sdb[j, 1, sl]

        def fire_scatters(p):
            for j in range(4):
                pltpu.async_copy(ones8, accd.at[dstbuf[p].at[j]],
                                 sems_sc[p], add=True)

        def drain_scatters(p):
            for j in range(4):
                pltpu.make_async_copy(ones8, accd.at[pl.ds(0, 128)],
                                      sems_sc[p]).wait()

        _edge_pipeline(NSB1, w * 200, sd_hbm, accd, (sdb0, sdb1), dstbuf,
                       (ix0, ix1), sems_sc, compute_dst, fire_scatters,
                       drain_scatters)

        nb = w * (NBK * 128)

        def node_blk(b, carry):
            pltpu.sync_copy(batch_hbm.at[pl.ds(nb + b * 128, 128)], dix)
            pltpu.sync_copy(ones_v, accc.at[dix], add=True)
            return carry

        lax.fori_loop(0, NBK, node_blk, 0)
        plsc.subcore_barrier()
        for step, off8 in ((784, 0), (784, 784), (784, 1568), (776, 2352)):
            pltpu.sync_copy(accd.at[pl.ds(s * ROW1 + off8, step)],
                            zbuf8.at[pl.ds(0, step)])
            pltpu.sync_copy(zbuf8.at[pl.ds(0, step)],
                            deg_out.at[c, pl.ds(s * ROW1 + off8, step)])
        pltpu.sync_copy(accc.at[pl.ds(s * ROWG, ROWG)], zbuf.at[pl.ds(0, ROWG)])
        pltpu.sync_copy(zbuf.at[pl.ds(0, ROWG)], cnt_out.at[c, pl.ds(s * ROWG, ROWG)])

    return k(sd, batch_p, ones_blk, ones8_blk, z16, z8)


def _conv_scratch():
    return [
        pltpu.VMEM((504, 16), jnp.float32),
        pltpu.VMEM((4, 2, 128), jnp.int32),
        pltpu.VMEM((4, 2, 128), jnp.int32),
        pltpu.VMEM((4, 128), jnp.int32),
        pltpu.VMEM((4, 128), jnp.int32),
        pltpu.VMEM((4, 128), jnp.int32),
        pltpu.VMEM((4, 128), jnp.int32),
        pltpu.VMEM((SB, 16), jnp.float32),
        pltpu.VMEM((SB, 16), jnp.float32),
        pltpu.SemaphoreType.DMA,
        pltpu.SemaphoreType.DMA,
        pltpu.SemaphoreType.DMA,
        pltpu.SemaphoreType.DMA,
        pltpu.SemaphoreType.DMA,
        pltpu.SemaphoreType.DMA,
        pltpu.VMEM_SHARED((NP, 16), jnp.float32),
    ]


def _make_conv_helpers(acc, dstbuf, rows, sems_sc):
    def compute_dst(db, sdb, j, sl):
        db[j, sl] = sdb[j, 1, sl]

    def fire_scatters(p):
        for j in range(4):
            pltpu.async_copy(rows[p].at[pl.ds(j * 128, 128)],
                             acc.at[dstbuf[p].at[j]], sems_sc[p], add=True)

    def drain_scatters(p):
        pltpu.make_async_copy(rows[p], acc.at[pl.ds(0, SB)], sems_sc[p]).wait()

    return compute_dst, fire_scatters, drain_scatters


def _edge_scatter16(tbl, sd, z16):
    @functools.partial(
        pl.kernel,
        out_type=jax.ShapeDtypeStruct((2, NP, 16), jnp.float32),
        mesh=_mesh(),
        compiler_params=_CP,
        scratch_types=_conv_scratch(),
    )
    def k(tbl_hbm, sd_hbm, z16_hbm, p_out,
          zbuf, sdb0, sdb1, fb0, fb1, db0, db1, rows0, rows1,
          ix0, ix1, g0, g1, sc0, sc1, acc):
        c = lax.axis_index("c")
        s = lax.axis_index("s")
        w = c * 16 + s
        pltpu.sync_copy(z16_hbm.at[pl.ds(0, 504)], zbuf)
        _zero_rows(zbuf, acc, s * ROW1, ROW1)
        plsc.subcore_barrier()

        dstbuf = (db0, db1)
        rows = (rows0, rows1)
        sems_sc = (sc0, sc1)
        compute_dst, fire_scatters, drain_scatters = _make_conv_helpers(
            acc, dstbuf, rows, sems_sc)

        _edge_pipeline(NSB1, w * 200, sd_hbm, acc, (sdb0, sdb1), dstbuf,
                       (ix0, ix1), sems_sc, compute_dst, fire_scatters,
                       drain_scatters, fixbuf=(fb0, fb1), rows=rows,
                       tbl_hbm=tbl_hbm, sems_g=(g0, g1),
                       off=jnp.int32(0))

        plsc.subcore_barrier()
        _copy_rows(acc, s * ROW1, ROW1, zbuf, p_out.at[c], s * ROW1)

    return k(tbl, sd, z16)


def _edge_scatter64(tbl4, sd, z16, disb, dh, batch_p):
    @functools.partial(
        pl.kernel,
        out_type=jax.ShapeDtypeStruct((4, GP, 16), jnp.float32),
        mesh=_mesh(),
        compiler_params=_CP,
        scratch_types=[
            pltpu.VMEM((1024, 16), jnp.float32),
            pltpu.VMEM((4, 2, 128), jnp.int32),
            pltpu.VMEM((4, 2, 128), jnp.int32),
            pltpu.VMEM((4, 128), jnp.int32),
            pltpu.VMEM((4, 128), jnp.int32),
            pltpu.VMEM((4, 128), jnp.int32),
            pltpu.VMEM((4, 128), jnp.int32),
            pltpu.VMEM((SB, 16), jnp.float32),
            pltpu.VMEM((SB, 16), jnp.float32),
            pltpu.VMEM((1024, 16), jnp.float32),
            pltpu.VMEM((1024, 16), jnp.float32),
            pltpu.VMEM((128,), jnp.int32),
            pltpu.VMEM((56,), jnp.int32),
            pltpu.SemaphoreType.DMA,
            pltpu.SemaphoreType.DMA,
            pltpu.SemaphoreType.DMA,
            pltpu.SemaphoreType.DMA,
            pltpu.SemaphoreType.DMA,
            pltpu.SemaphoreType.DMA,
            pltpu.VMEM_SHARED((NP, 16), jnp.float32),
            pltpu.VMEM_SHARED((GP, 16), jnp.float32),
            pltpu.VMEM_SHARED((GP, 16), jnp.float32),
        ],
    )
    def k(tbl_hbm, sd_hbm, z16_hbm, disb_hbm, dh_hbm, batch_hbm, pool_out,
          zbuf, sdb0, sdb1, fb0, fb1, db0, db1, rows0, rows1, disbuf, hbuf,
          dix, dix56, ix0, ix1, g0, g1, sc0, sc1, acc, accp0, accp1):
        c = lax.axis_index("c")
        s = lax.axis_index("s")

        dstbuf = (db0, db1)
        rows = (rows0, rows1)
        sems_sc = (sc0, sc1)
        accps = (accp0, accp1)
        compute_dst, fire_scatters, drain_scatters = _make_conv_helpers(
            acc, dstbuf, rows, sems_sc)

        pltpu.sync_copy(z16_hbm, zbuf)
        for q in range(2):
            pltpu.sync_copy(zbuf.at[pl.ds(0, ROWG)],
                            accps[q].at[pl.ds(s * ROWG, ROWG)])

        for q in range(2):
            chunk = 2 * c + q
            pltpu.sync_copy(z16_hbm, zbuf)
            _zero_rows(zbuf, acc, s * ROW1, ROW1)
            plsc.subcore_barrier()

            _edge_pipeline(NSB2, s * 400, sd_hbm, acc, (sdb0, sdb1), dstbuf,
                           (ix0, ix1), sems_sc, compute_dst, fire_scatters,
                           drain_scatters, fixbuf=(fb0, fb1), rows=rows,
                           tbl_hbm=tbl_hbm, sems_g=(g0, g1),
                           off=chunk * NP)

            plsc.subcore_barrier()
            node00 = s * ROW1
            done = 0
            for step in (1024, 1024, 1024, 56):
                base = node00 + done
                pltpu.sync_copy(acc.at[pl.ds(base, step)],
                                zbuf.at[pl.ds(0, step)])
                pltpu.sync_copy(disb_hbm.at[pl.ds(base, step)],
                                disbuf.at[pl.ds(0, step)])
                pltpu.sync_copy(dh_hbm.at[chunk, pl.ds(base, step)],
                                hbuf.at[pl.ds(0, step)])

                def zrow(r, carry):
                    zbuf[r, :] = zbuf[r, :] * disbuf[r, :] + hbuf[r, :]
                    return carry

                lax.fori_loop(0, step, zrow, 0)
                for b in range(step // 128):
                    pltpu.sync_copy(batch_hbm.at[pl.ds(base + b * 128, 128)],
                                    dix)
                    pltpu.sync_copy(zbuf.at[pl.ds(b * 128, 128)],
                                    accps[q].at[dix], add=True)
                if step % 128:
                    tail = step % 128
                    toff = (step // 128) * 128
                    pltpu.sync_copy(batch_hbm.at[pl.ds(base + toff, tail)],
                                    dix56)
                    pltpu.sync_copy(zbuf.at[pl.ds(toff, tail)],
                                    accps[q].at[dix56], add=True)
                done += step
            plsc.subcore_barrier()

        for q in range(2):
            pltpu.sync_copy(accps[q].at[pl.ds(s * ROWG, ROWG)],
                            zbuf.at[pl.ds(0, ROWG)])
            pltpu.sync_copy(zbuf.at[pl.ds(0, ROWG)],
                            pool_out.at[2 * c + q, pl.ds(s * ROWG, ROWG)])

    return k(tbl4, sd, z16, disb, dh, batch_p)


def kernel(x, edge_index, batch, W1, b1, W2, b2, Wl, bl):
    src = edge_index[0].astype(jnp.int32)
    dst = edge_index[1].astype(jnp.int32)
    batch = batch.astype(jnp.int32)

    src_p = jnp.concatenate([src, jnp.zeros((EP - E,), jnp.int32)])
    dst_p = jnp.concatenate([dst, jnp.full((EP - E,), N, jnp.int32)])
    sd = jnp.stack([src_p.reshape(-1, 128), dst_p.reshape(-1, 128)], axis=1)
    batch_p = jnp.concatenate([batch, jnp.full((NB - N,), G, jnp.int32)])
    ones_blk = jnp.ones((128, 16), jnp.float32)
    ones8_blk = jnp.ones((128, 8), jnp.float32)
    z16 = jnp.zeros((1000, 16), jnp.float32)
    z16p = jnp.zeros((1024, 16), jnp.float32)
    z8 = jnp.zeros((784, 8), jnp.float32)

    deg_p, cnt_p = _deg_counts(sd, batch_p, ones_blk, ones8_blk, z16, z8)
    deg = deg_p[0, :N, 0] + deg_p[1, :N, 0] + 1.0
    counts = cnt_p[0, :G, 0] + cnt_p[1, :G, 0]
    dis = deg ** -0.5
    dis2 = dis * dis

    y0 = jnp.pad(x * dis[:, None], ((0, NP - N), (0, 16 - x.shape[1])))
    p0 = _edge_scatter16(y0, sd, z16)
    s0 = (p0[0] + p0[1])[:N, : x.shape[1]]
    z1 = dis[:, None] * s0 + dis2[:, None] * x

    h1 = jax.nn.relu(z1 @ W1 + b1)
    y1 = jnp.pad(h1 * dis[:, None], ((0, NP - N), (0, 0)))
    tbl4 = y1.reshape(NP, 4, 16).transpose(1, 0, 2).reshape(4 * NP, 16)
    disb = jnp.pad(jnp.broadcast_to(dis[:, None], (N, 16)), ((0, NP - N), (0, 0)))
    dh = jnp.pad(dis2[:, None] * h1, ((0, NP - N), (0, 0)))
    dh = dh.reshape(NP, 4, 16).transpose(1, 0, 2)
    pools = _edge_scatter64(tbl4, sd, z16p, disb, dh, batch_p)

    sums = jnp.concatenate([pools[k, :G] for k in range(4)], axis=1)
    g_pre = sums / jnp.clip(counts, 1.0)[:, None]
    out = g_pre @ (W2 @ Wl) + (counts > 0.0)[:, None] * (b2 @ Wl) + bl
    return out


# speedup vs baseline: 1.5820x; 1.0013x over previous
import functools

import jax
import jax.numpy as jnp
from jax import lax
from jax.experimental import pallas as pl
from jax.experimental.pallas import tpu as pltpu
from jax.experimental.pallas import tpu_sc as plsc

N = 50000
E = 800000
G = 1024
NP = 50048
EP = 819200
SB = 512
NB = 53248
GP = 1152
ROW1 = 3128
ROWG = 72
NSB1 = 50
NSB2 = 100
NBK = 13
NBK2 = 26


def _mesh():
    return plsc.VectorSubcoreMesh(core_axis_name="c", subcore_axis_name="s")


_CP = pltpu.CompilerParams(use_tc_tiling_on_sc=False)


def _zero_rows(zbuf, acc, base, nrows):
    cap = zbuf.shape[0]
    done = 0
    while done < nrows:
        step = min(cap, nrows - done)
        pltpu.sync_copy(zbuf.at[pl.ds(0, step)], acc.at[pl.ds(base + done, step)])
        done += step


def _copy_rows(acc, base, nrows, zbuf, out_ref, obase):
    cap = zbuf.shape[0]
    done = 0
    while done < nrows:
        step = min(cap, nrows - done)
        pltpu.sync_copy(acc.at[pl.ds(base + done, step)], zbuf.at[pl.ds(0, step)])
        pltpu.sync_copy(zbuf.at[pl.ds(0, step)], out_ref.at[pl.ds(obase + done, step)])
        done += step


def _edge_pipeline(nsb, brow0, sd_hbm, acc, sdbuf, dstbuf, sems_ix, sems_sc,
                   compute_dst, fire_scatters, drain_scatters,
                   fixbuf=None, rows=None, tbl_hbm=None, sems_g=None, off=None):
    gather = tbl_hbm is not None

    def idx_start(c, p):
        pltpu.async_copy(sd_hbm.at[pl.ds(brow0 + c * 4, 4)], sdbuf[p], sems_ix[p])

    def idx_drain(p):
        pltpu.make_async_copy(sd_hbm.at[pl.ds(brow0, 4)], sdbuf[p], sems_ix[p]).wait()

    def compute(p):
        for j in range(4):
            for kk in range(8):
                sl = pl.ds(kk * 16, 16)
                if gather:
                    fixbuf[p][j, sl] = sdbuf[p][j, 0, sl] + off
                compute_dst(dstbuf[p], sdbuf[p], j, sl)

    def gather_fire(p):
        for j in range(4):
            pltpu.async_copy(tbl_hbm.at[fixbuf[p].at[j]],
                             rows[p].at[pl.ds(j * 128, 128)], sems_g[p])

    def gather_drain(p):
        pltpu.make_async_copy(tbl_hbm.at[pl.ds(0, SB)], rows[p], sems_g[p]).wait()

    def stage_front(i, c, p):
        @pl.when(i > 0)
        def _():
            drain_scatters(p)

        idx_drain(p)
        compute(p)

        @pl.when(c + 2 < nsb)
        def _():
            idx_start(c + 2, p)

        if gather:
            gather_fire(p)

    def body(i, carry):
        a = 2 * i
        stage_front(i, a, 0)
        stage_front(i, a + 1, 1)
        if gather:
            gather_drain(0)
        fire_scatters(0)
        if gather:
            gather_drain(1)
        fire_scatters(1)
        return carry

    idx_start(0, 0)
    idx_start(1, 1)
    lax.fori_loop(0, nsb // 2, body, 0)
    drain_scatters(0)
    drain_scatters(1)


def _deg_counts(sd, batch_p, ones_blk, ones8_blk, z16, z8):
    @functools.partial(
        pl.kernel,
        out_type=(
            jax.ShapeDtypeStruct((2, NP, 8), jnp.float32),
            jax.ShapeDtypeStruct((2, GP, 16), jnp.float32),
        ),
        mesh=_mesh(),
        compiler_params=_CP,
        scratch_types=[
            pltpu.VMEM((504, 16), jnp.float32),
            pltpu.VMEM((784, 8), jnp.float32),
            pltpu.VMEM((128, 16), jnp.float32),
            pltpu.VMEM((128, 8), jnp.float32),
            pltpu.VMEM((128,), jnp.int32),
            pltpu.VMEM((4, 2, 128), jnp.int32),
            pltpu.VMEM((4, 2, 128), jnp.int32),
            pltpu.VMEM((4, 128), jnp.int32),
            pltpu.VMEM((4, 128), jnp.int32),
            pltpu.SemaphoreType.DMA,
            pltpu.SemaphoreType.DMA,
            pltpu.SemaphoreType.DMA,
            pltpu.SemaphoreType.DMA,
            pltpu.VMEM_SHARED((NP, 8), jnp.float32),
            pltpu.VMEM_SHARED((GP, 16), jnp.float32),
        ],
    )
    def k(sd_hbm, batch_hbm, ones_hbm, ones8_hbm, z16_hbm, z8_hbm,
          deg_out, cnt_out,
          zbuf, zbuf8, ones_v, ones8, dix, sdb0, sdb1, db0, db1,
          ix0, ix1, sc0, sc1, accd, accc):
        c = lax.axis_index("c")
        s = lax.axis_index("s")
        w = c * 16 + s
        pltpu.sync_copy(z16_hbm.at[pl.ds(0, 504)], zbuf)
        pltpu.sync_copy(z8_hbm.at[pl.ds(0, 784)], zbuf8)
        for step, off8 in ((784, 0), (784, 784), (784, 1568), (776, 2352)):
            pltpu.sync_copy(zbuf8.at[pl.ds(0, step)],
                            accd.at[pl.ds(s * ROW1 + off8, step)])
        pltpu.sync_copy(zbuf.at[pl.ds(0, ROWG)], accc.at[pl.ds(s * ROWG, ROWG)])
        pltpu.sync_copy(ones_hbm, ones_v)
        pltpu.sync_copy(ones8_hbm, ones8)
        plsc.subcore_barrier()

        dstbuf = (db0, db1)
        sems_sc = (sc0, sc1)

        def compute_dst(db, sdb, j, sl):
            db[j, sl] = sdb[j, 1, sl]

        def fire_scatters(p):
            for j in range(4):
                pltpu.async_copy(ones8, accd.at[dstbuf[p].at[j]],
                                 sems_sc[p], add=True)

        def drain_scatters(p):
            for j in range(4):
                pltpu.make_async_copy(ones8, accd.at[pl.ds(0, 128)],
                                      sems_sc[p]).wait()

        _edge_pipeline(NSB1, w * 200, sd_hbm, accd, (sdb0, sdb1), dstbuf,
                       (ix0, ix1), sems_sc, compute_dst, fire_scatters,
                       drain_scatters)

        nb = w * (NBK * 128)

        def node_blk(b, carry):
            pltpu.sync_copy(batch_hbm.at[pl.ds(nb + b * 128, 128)], dix)
            pltpu.sync_copy(ones_v, accc.at[dix], add=True)
            return carry

        lax.fori_loop(0, NBK, node_blk, 0)
        plsc.subcore_barrier()
        for step, off8 in ((784, 0), (784, 784), (784, 1568), (776, 2352)):
            pltpu.sync_copy(accd.at[pl.ds(s * ROW1 + off8, step)],
                            zbuf8.at[pl.ds(0, step)])
            pltpu.sync_copy(zbuf8.at[pl.ds(0, step)],
                            deg_out.at[c, pl.ds(s * ROW1 + off8, step)])
        pltpu.sync_copy(accc.at[pl.ds(s * ROWG, ROWG)], zbuf.at[pl.ds(0, ROWG)])
        pltpu.sync_copy(zbuf.at[pl.ds(0, ROWG)], cnt_out.at[c, pl.ds(s * ROWG, ROWG)])

    return k(sd, batch_p, ones_blk, ones8_blk, z16, z8)


def _conv_scratch():
    return [
        pltpu.VMEM((504, 16), jnp.float32),
        pltpu.VMEM((4, 2, 128), jnp.int32),
        pltpu.VMEM((4, 2, 128), jnp.int32),
        pltpu.VMEM((4, 128), jnp.int32),
        pltpu.VMEM((4, 128), jnp.int32),
        pltpu.VMEM((4, 128), jnp.int32),
        pltpu.VMEM((4, 128), jnp.int32),
        pltpu.VMEM((SB, 16), jnp.float32),
        pltpu.VMEM((SB, 16), jnp.float32),
        pltpu.SemaphoreType.DMA,
        pltpu.SemaphoreType.DMA,
        pltpu.SemaphoreType.DMA,
        pltpu.SemaphoreType.DMA,
        pltpu.SemaphoreType.DMA,
        pltpu.SemaphoreType.DMA,
        pltpu.VMEM_SHARED((NP, 16), jnp.float32),
    ]


def _make_conv_helpers(acc, dstbuf, rows, sems_sc):
    def compute_dst(db, sdb, j, sl):
        db[j, sl] = sdb[j, 1, sl]

    def fire_scatters(p):
        for j in range(4):
            pltpu.async_copy(rows[p].at[pl.ds(j * 128, 128)],
                             acc.at[dstbuf[p].at[j]], sems_sc[p], add=True)

    def drain_scatters(p):
        pltpu.make_async_copy(rows[p], acc.at[pl.ds(0, SB)], sems_sc[p]).wait()

    return compute_dst, fire_scatters, drain_scatters


def _edge_scatter16(tbl, sd, z16):
    @functools.partial(
        pl.kernel,
        out_type=jax.ShapeDtypeStruct((2, NP, 16), jnp.float32),
        mesh=_mesh(),
        compiler_params=_CP,
        scratch_types=_conv_scratch(),
    )
    def k(tbl_hbm, sd_hbm, z16_hbm, p_out,
          zbuf, sdb0, sdb1, fb0, fb1, db0, db1, rows0, rows1,
          ix0, ix1, g0, g1, sc0, sc1, acc):
        c = lax.axis_index("c")
        s = lax.axis_index("s")
        w = c * 16 + s
        pltpu.sync_copy(z16_hbm.at[pl.ds(0, 504)], zbuf)
        _zero_rows(zbuf, acc, s * ROW1, ROW1)
        plsc.subcore_barrier()

        dstbuf = (db0, db1)
        rows = (rows0, rows1)
        sems_sc = (sc0, sc1)
        compute_dst, fire_scatters, drain_scatters = _make_conv_helpers(
            acc, dstbuf, rows, sems_sc)

        _edge_pipeline(NSB1, w * 200, sd_hbm, acc, (sdb0, sdb1), dstbuf,
                       (ix0, ix1), sems_sc, compute_dst, fire_scatters,
                       drain_scatters, fixbuf=(fb0, fb1), rows=rows,
                       tbl_hbm=tbl_hbm, sems_g=(g0, g1),
                       off=jnp.int32(0))

        plsc.subcore_barrier()
        _copy_rows(acc, s * ROW1, ROW1, zbuf, p_out.at[c], s * ROW1)

    return k(tbl, sd, z16)


def _edge_scatter64(tbl4, sd, z16, disb, dh, batch_p):
    @functools.partial(
        pl.kernel,
        out_type=jax.ShapeDtypeStruct((4, GP, 16), jnp.float32),
        mesh=_mesh(),
        compiler_params=_CP,
        scratch_types=[
            pltpu.VMEM((1024, 16), jnp.float32),
            pltpu.VMEM((4, 2, 128), jnp.int32),
            pltpu.VMEM((4, 2, 128), jnp.int32),
            pltpu.VMEM((4, 128), jnp.int32),
            pltpu.VMEM((4, 128), jnp.int32),
            pltpu.VMEM((4, 128), jnp.int32),
            pltpu.VMEM((4, 128), jnp.int32),
            pltpu.VMEM((SB, 16), jnp.float32),
            pltpu.VMEM((SB, 16), jnp.float32),
            pltpu.VMEM((1024, 16), jnp.float32),
            pltpu.VMEM((1024, 16), jnp.float32),
            pltpu.VMEM((128,), jnp.int32),
            pltpu.VMEM((56,), jnp.int32),
            pltpu.SemaphoreType.DMA,
            pltpu.SemaphoreType.DMA,
            pltpu.SemaphoreType.DMA,
            pltpu.SemaphoreType.DMA,
            pltpu.SemaphoreType.DMA,
            pltpu.SemaphoreType.DMA,
            pltpu.VMEM_SHARED((NP, 16), jnp.float32),
            pltpu.VMEM_SHARED((GP, 16), jnp.float32),
            pltpu.VMEM_SHARED((GP, 16), jnp.float32),
        ],
    )
    def k(tbl_hbm, sd_hbm, z16_hbm, disb_hbm, dh_hbm, batch_hbm, pool_out,
          zbuf, sdb0, sdb1, fb0, fb1, db0, db1, rows0, rows1, disbuf, hbuf,
          dix, dix56, ix0, ix1, g0, g1, sc0, sc1, acc, accp0, accp1):
        c = lax.axis_index("c")
        s = lax.axis_index("s")

        dstbuf = (db0, db1)
        rows = (rows0, rows1)
        sems_sc = (sc0, sc1)
        accps = (accp0, accp1)
        compute_dst, fire_scatters, drain_scatters = _make_conv_helpers(
            acc, dstbuf, rows, sems_sc)

        pltpu.sync_copy(z16_hbm, zbuf)
        for q in range(2):
            pltpu.sync_copy(zbuf.at[pl.ds(0, ROWG)],
                            accps[q].at[pl.ds(s * ROWG, ROWG)])

        for q in range(2):
            chunk = 2 * c + q
            pltpu.sync_copy(z16_hbm, zbuf)
            _zero_rows(zbuf, acc, s * ROW1, ROW1)
            plsc.subcore_barrier()

            _edge_pipeline(NSB2, s * 400, sd_hbm, acc, (sdb0, sdb1), dstbuf,
                           (ix0, ix1), sems_sc, compute_dst, fire_scatters,
                           drain_scatters, fixbuf=(fb0, fb1), rows=rows,
                           tbl_hbm=tbl_hbm, sems_g=(g0, g1),
                           off=chunk * NP)

            plsc.subcore_barrier()
            node00 = s * ROW1
            done = 0
            for step in (1024, 1024, 1024, 56):
                base = node00 + done
                pltpu.sync_copy(acc.at[pl.ds(base, step)],
                                zbuf.at[pl.ds(0, step)])
                pltpu.sync_copy(disb_hbm.at[pl.ds(base, step)],
                                disbuf.at[pl.ds(0, step)])
                pltpu.sync_copy(dh_hbm.at[chunk, pl.ds(base, step)],
                                hbuf.at[pl.ds(0, step)])

                def zrow(r, carry):
                    zbuf[r, :] = zbuf[r, :] * disbuf[r, :] + hbuf[r, :]
                    return carry

                lax.fori_loop(0, step, zrow, 0)
                for b in range(step // 128):
                    pltpu.sync_copy(batch_hbm.at[pl.ds(base + b * 128, 128)],
                                    dix)
                    pltpu.sync_copy(zbuf.at[pl.ds(b * 128, 128)],
                                    accps[q].at[dix], add=True)
                if step % 128:
                    tail = step % 128
                    toff = (step // 128) * 128
                    pltpu.sync_copy(batch_hbm.at[pl.ds(base + toff, tail)],
                                    dix56)
                    pltpu.sync_copy(zbuf.at[pl.ds(toff, tail)],
                                    accps[q].at[dix56], add=True)
                done += step
            plsc.subcore_barrier()

        for q in range(2):
            pltpu.sync_copy(accps[q].at[pl.ds(s * ROWG, ROWG)],
                            zbuf.at[pl.ds(0, ROWG)])
            pltpu.sync_copy(zbuf.at[pl.ds(0, ROWG)],
                            pool_out.at[2 * c + q, pl.ds(s * ROWG, ROWG)])

    return k(tbl4, sd, z16, disb, dh, batch_p)


def kernel(x, edge_index, batch, W1, b1, W2, b2, Wl, bl):
    src = edge_index[0].astype(jnp.int32)
    dst = edge_index[1].astype(jnp.int32)
    batch = batch.astype(jnp.int32)

    src_p = jnp.concatenate([src, jnp.zeros((EP - E,), jnp.int32)])
    dst_p = jnp.concatenate([dst, jnp.full((EP - E,), N, jnp.int32)])
    sd = jnp.stack([src_p.reshape(-1, 128), dst_p.reshape(-1, 128)], axis=1)
    batch_p = jnp.concatenate([batch, jnp.full((NB - N,), G, jnp.int32)])
    ones_blk = jnp.ones((128, 16), jnp.float32)
    ones8_blk = jnp.ones((128, 8), jnp.float32)
    z16 = jnp.zeros((1000, 16), jnp.float32)
    z16p = jnp.zeros((1024, 16), jnp.float32)
    z8 = jnp.zeros((784, 8), jnp.float32)

    deg_p, cnt_p = _deg_counts(sd, batch_p, ones_blk, ones8_blk, z16, z8)
    deg = deg_p[0, :N, 0] + deg_p[1, :N, 0] + 1.0
    counts = cnt_p[0, :G, 0] + cnt_p[1, :G, 0]
    dis = deg ** -0.5
    dis2 = dis * dis

    y0 = jnp.pad(x * dis[:, None], ((0, NP - N), (0, 16 - x.shape[1])))
    p0 = _edge_scatter16(y0, sd, z16)
    s0 = (p0[0] + p0[1])[:N, : x.shape[1]]
    z1 = dis[:, None] * s0 + dis2[:, None] * x

    h1 = jax.nn.relu(z1 @ W1 + b1)
    y1 = jnp.pad(h1 * dis[:, None], ((0, NP - N), (0, 0)))
    tbl4 = y1.reshape(NP, 4, 16).transpose(1, 0, 2).reshape(4 * NP, 16)
    disb = jnp.pad(jnp.broadcast_to(dis[:, None], (N, 16)), ((0, NP - N), (0, 0)))
    dh = jnp.pad(dis2[:, None] * h1, ((0, NP - N), (0, 0)))
    dh = dh.reshape(NP, 4, 16).transpose(1, 0, 2)
    pools = _edge_scatter64(tbl4, sd, z16p, disb, dh, batch_p)

    sums = jnp.concatenate([pools[k, :G] for k in range(4)], axis=1)
    g_pre = sums / jnp.clip(counts, 1.0)[:, None]
    out = g_pre @ (W2 @ Wl) + (counts > 0.0)[:, None] * (b2 @ Wl) + bl
    return out
